# Initial kernel scaffold; baseline (speedup 1.0000x reference)
import jax
import jax.numpy as jnp
from jax.experimental import pallas as pl


def kernel(sequence_tensor, positions):
    raise NotImplementedError("write your pallas kernel here")


# speedup vs baseline: 1.4126x; 1.4126x over previous
import functools

import jax
import jax.numpy as jnp
from jax import lax
from jax.experimental import pallas as pl
from jax.experimental.pallas import tpu as pltpu
from jax.experimental.pallas import tpu_sc as plsc

_LANES = 16


def _gather_kernel(num_rows, width, npos, seq_len, n_workers):
    b_per_w = num_rows // n_workers
    mesh = plsc.VectorSubcoreMesh(core_axis_name="c", subcore_axis_name="s")

    @functools.partial(
        pl.kernel,
        mesh=mesh,
        out_type=jax.ShapeDtypeStruct((num_rows, width), jnp.float32),
        scratch_types=[
            pltpu.VMEM((b_per_w,), jnp.int32),
            pltpu.VMEM((b_per_w, width), jnp.float32),
            pltpu.SemaphoreType.DMA,
        ],
    )
    def k(table_hbm, idx_hbm, out_hbm, idx_v, rows_v, sem):
        wid = lax.axis_index("s") * 2 + lax.axis_index("c")
        base = wid * b_per_w
        pltpu.sync_copy(idx_hbm.at[pl.ds(base, b_per_w)], idx_v)
        off = (base // npos) * seq_len
        for i in range(b_per_w // _LANES):
            sl = pl.ds(i * _LANES, _LANES)
            idx_v[sl] = idx_v[sl] + off
        pltpu.async_copy(table_hbm.at[idx_v], rows_v, sem).wait()
        pltpu.sync_copy(rows_v, out_hbm.at[pl.ds(base, b_per_w)])

    return k


def kernel(sequence_tensor, positions):
    batch, seq_len, width = sequence_tensor.shape
    nb, npos = positions.shape
    num_rows = nb * npos
    table = sequence_tensor.reshape(batch * seq_len, width)
    idx = positions.reshape(num_rows).astype(jnp.int32)
    n_workers = 32
    assert num_rows % n_workers == 0
    assert npos % (num_rows // n_workers) == 0
    k = _gather_kernel(num_rows, width, npos, seq_len, n_workers)
    return k(table, idx)


# speedup vs baseline: 1.4142x; 1.0011x over previous
import functools

import jax
import jax.numpy as jnp
from jax import lax
from jax.experimental import pallas as pl
from jax.experimental.pallas import tpu as pltpu
from jax.experimental.pallas import tpu_sc as plsc

_LANES = 16


def _gather_kernel(num_rows, width, npos, seq_len, n_workers, n_chunks):
    b_per_w = num_rows // n_workers
    rows_c = b_per_w // n_chunks
    assert rows_c % 8 == 0
    mesh = plsc.VectorSubcoreMesh(core_axis_name="c", subcore_axis_name="s")

    @functools.partial(
        pl.kernel,
        mesh=mesh,
        out_type=jax.ShapeDtypeStruct((num_rows, width), jnp.float32),
        scratch_types=[
            pltpu.VMEM((b_per_w,), jnp.int32),
            pltpu.VMEM((b_per_w, width), jnp.float32),
        ]
        + [pltpu.SemaphoreType.DMA] * (n_chunks + 1),
    )
    def k(table_hbm, idx_hbm, out_hbm, idx_v, rows_v, *sems):
        gsems, wsem = sems[:n_chunks], sems[n_chunks]
        wid = lax.axis_index("s") * 2 + lax.axis_index("c")
        base = wid * b_per_w
        pltpu.sync_copy(idx_hbm.at[pl.ds(base, b_per_w)], idx_v)
        off = (base // npos) * seq_len
        for i in range(b_per_w // _LANES):
            sl = pl.ds(i * _LANES, _LANES)
            idx_v[sl] = idx_v[sl] + off
        gh = [
            pltpu.async_copy(
                table_hbm.at[idx_v.at[pl.ds(c * rows_c, rows_c)]],
                rows_v.at[pl.ds(c * rows_c, rows_c)],
                gsems[c],
            )
            for c in range(n_chunks)
        ]
        wh = []
        for c in range(n_chunks):
            gh[c].wait()
            wh.append(
                pltpu.async_copy(
                    rows_v.at[pl.ds(c * rows_c, rows_c)],
                    out_hbm.at[pl.ds(base + c * rows_c, rows_c)],
                    wsem,
                )
            )
        for h in wh:
            h.wait()

    return k


def kernel(sequence_tensor, positions):
    batch, seq_len, width = sequence_tensor.shape
    nb, npos = positions.shape
    num_rows = nb * npos
    table = sequence_tensor.reshape(batch * seq_len, width)
    idx = positions.reshape(num_rows).astype(jnp.int32)
    n_workers = 32
    assert num_rows % n_workers == 0
    assert npos % (num_rows // n_workers) == 0
    k = _gather_kernel(num_rows, width, npos, seq_len, n_workers, n_chunks=5)
    return k(table, idx)
